# baseline (speedup 1.0000x reference)
import jax
import jax.numpy as jnp
from jax import lax
from jax.experimental import pallas as pl
from jax.experimental.pallas import tpu as pltpu

_E_BLOCK = 4000
_N_BLOCK = 2000


def _colsum_body(x1_ref, out_ref):
    i = pl.program_id(0)

    @pl.when(i == 0)
    def _init():
        out_ref[...] = jnp.zeros_like(out_ref)

    out_ref[0:1, :] += jnp.sum(x1_ref[...], axis=0, keepdims=True)


def _dense_body(s_ref, x0_ref, w0cat_ref, w1on_ref, w1tg_ref, p1w_ref,
                p1b_ref, a_ref, p2w_ref, p2b_ref, inv_e_ref,
                on_ref, pred_ref, tg_ref):
    def dot_t(a, b):
        return lax.dot_general(a, b, (((1,), (1,)), ((), ())),
                               preferred_element_type=jnp.float32)

    s128 = s_ref[0:1, :]
    m32 = (s128[:, 0:32] + s128[:, 32:64] + s128[:, 64:96]
           + s128[:, 96:128]) * inv_e_ref[0, 0]
    m_on = dot_t(m32, w1on_ref[...])
    m_tg = dot_t(m32, w1tg_ref[...])

    x0 = x0_ref[...]
    x0_cat = dot_t(x0, w0cat_ref[...])
    n = x0.shape[0]
    h_on = jnp.concatenate(
        [x0_cat[:, 0:32], jnp.broadcast_to(m_on, (n, 32))], axis=1)
    h_tg = jnp.concatenate(
        [x0_cat[:, 32:64], jnp.broadcast_to(m_tg, (n, 32))], axis=1)
    on_ref[...] = h_on
    tg_ref[...] = h_tg

    z = dot_t(h_on, p1w_ref[...]) + p1b_ref[...]
    a = a_ref[0, 0]
    h = jnp.where(z >= 0, z, a * z)
    pred_ref[...] = dot_t(h, p2w_ref[...]) + p2b_ref[...]


def kernel(x_0, x_1, adjacency_0, down_laplacian, up_laplacian,
           W0_on, W1_on, W0_tg, W1_tg, p1_w, p1_b, prelu_a, p2_w, p2_b):
    n, in0 = x_0.shape
    e, h = x_1.shape

    x1v = x_1.reshape(e // 4, 4 * h)
    n_red_blocks = (x1v.shape[0] + _E_BLOCK - 1) // _E_BLOCK

    sums = pl.pallas_call(
        _colsum_body,
        grid=(n_red_blocks,),
        in_specs=[pl.BlockSpec((_E_BLOCK, 4 * h), lambda i: (i, 0))],
        out_specs=pl.BlockSpec((8, 4 * h), lambda i: (0, 0)),
        out_shape=jax.ShapeDtypeStruct((8, 4 * h), jnp.float32),
    )(x1v)

    w0cat = jnp.concatenate([W0_on, W0_tg], axis=0)
    p1b = p1_b.reshape(1, h)
    p2b = p2_b.reshape(1, 2 * h)
    a = jnp.reshape(prelu_a, (1, 1))
    inv_e = jnp.full((1, 1), 1.0 / e, dtype=jnp.float32)

    const = lambda shape: pl.BlockSpec(shape, lambda i: tuple(0 for _ in shape))
    n_blocks = n // _N_BLOCK
    h_on, h_pred, h_tg = pl.pallas_call(
        _dense_body,
        grid=(n_blocks,),
        in_specs=[
            const((8, 4 * h)),
            pl.BlockSpec((_N_BLOCK, in0), lambda i: (i, 0)),
            const((2 * h, in0)),
            const((h, h)),
            const((h, h)),
            const((h, 2 * h)),
            const((1, h)),
            const((1, 1)),
            const((2 * h, h)),
            const((1, 2 * h)),
            const((1, 1)),
        ],
        out_specs=[
            pl.BlockSpec((_N_BLOCK, 2 * h), lambda i: (i, 0)),
            pl.BlockSpec((_N_BLOCK, 2 * h), lambda i: (i, 0)),
            pl.BlockSpec((_N_BLOCK, 2 * h), lambda i: (i, 0)),
        ],
        out_shape=[
            jax.ShapeDtypeStruct((n, 2 * h), jnp.float32),
            jax.ShapeDtypeStruct((n, 2 * h), jnp.float32),
            jax.ShapeDtypeStruct((n, 2 * h), jnp.float32),
        ],
    )(sums, x_0, w0cat, W1_on, W1_tg, p1_w, p1b, a, p2_w, p2b, inv_e)

    return (h_on, h_pred, h_tg)


# speedup vs baseline: 1.1294x; 1.1294x over previous
import jax
import jax.numpy as jnp
from jax import lax
from jax.experimental import pallas as pl
from jax.experimental.pallas import tpu as pltpu

_E_BLOCK = 16000
_N_BLOCK = 2000


def _colsum_body(x1_ref, out_ref):
    i = pl.program_id(0)

    @pl.when(i == 0)
    def _init():
        out_ref[...] = jnp.zeros_like(out_ref)

    x = x1_ref[...]
    rows = x.shape[0]
    seg = rows // 8
    parts = [jnp.sum(x[k * seg:(k + 1) * seg, :], axis=0, keepdims=True)
             for k in range(8)]
    p0 = (parts[0] + parts[1]) + (parts[2] + parts[3])
    p1 = (parts[4] + parts[5]) + (parts[6] + parts[7])
    out_ref[0:1, :] += p0 + p1


def _dense_body(s_ref, x0_ref, w0cat_ref, w1on_ref, w1tg_ref, p1w_ref,
                p1b_ref, a_ref, p2w_ref, p2b_ref, inv_e_ref,
                on_ref, pred_ref, tg_ref):
    def dot_t(a, b):
        return lax.dot_general(a, b, (((1,), (1,)), ((), ())),
                               preferred_element_type=jnp.float32)

    m32 = s_ref[0:1, :] * inv_e_ref[0, 0]
    m_on = dot_t(m32, w1on_ref[...])
    m_tg = dot_t(m32, w1tg_ref[...])

    x0 = x0_ref[...]
    x0_cat = dot_t(x0, w0cat_ref[...])
    n = x0.shape[0]
    h_on = jnp.concatenate(
        [x0_cat[:, 0:32], jnp.broadcast_to(m_on, (n, 32))], axis=1)
    h_tg = jnp.concatenate(
        [x0_cat[:, 32:64], jnp.broadcast_to(m_tg, (n, 32))], axis=1)
    on_ref[...] = h_on
    tg_ref[...] = h_tg

    z = dot_t(h_on, p1w_ref[...]) + p1b_ref[...]
    a = a_ref[0, 0]
    h = jnp.where(z >= 0, z, a * z)
    pred_ref[...] = dot_t(h, p2w_ref[...]) + p2b_ref[...]


def kernel(x_0, x_1, adjacency_0, down_laplacian, up_laplacian,
           W0_on, W1_on, W0_tg, W1_tg, p1_w, p1_b, prelu_a, p2_w, p2_b):
    n, in0 = x_0.shape
    e, h = x_1.shape

    n_red_blocks = e // _E_BLOCK

    sums = pl.pallas_call(
        _colsum_body,
        grid=(n_red_blocks,),
        in_specs=[pl.BlockSpec((_E_BLOCK, h), lambda i: (i, 0))],
        out_specs=pl.BlockSpec((8, h), lambda i: (0, 0)),
        out_shape=jax.ShapeDtypeStruct((8, h), jnp.float32),
    )(x_1)

    w0cat = jnp.concatenate([W0_on, W0_tg], axis=0)
    p1b = p1_b.reshape(1, h)
    p2b = p2_b.reshape(1, 2 * h)
    a = jnp.reshape(prelu_a, (1, 1))
    inv_e = jnp.full((1, 1), 1.0 / e, dtype=jnp.float32)

    const = lambda shape: pl.BlockSpec(shape, lambda i: tuple(0 for _ in shape))
    n_blocks = n // _N_BLOCK
    h_on, h_pred, h_tg = pl.pallas_call(
        _dense_body,
        grid=(n_blocks,),
        in_specs=[
            const((8, h)),
            pl.BlockSpec((_N_BLOCK, in0), lambda i: (i, 0)),
            const((2 * h, in0)),
            const((h, h)),
            const((h, h)),
            const((h, 2 * h)),
            const((1, h)),
            const((1, 1)),
            const((2 * h, h)),
            const((1, 2 * h)),
            const((1, 1)),
        ],
        out_specs=[
            pl.BlockSpec((_N_BLOCK, 2 * h), lambda i: (i, 0)),
            pl.BlockSpec((_N_BLOCK, 2 * h), lambda i: (i, 0)),
            pl.BlockSpec((_N_BLOCK, 2 * h), lambda i: (i, 0)),
        ],
        out_shape=[
            jax.ShapeDtypeStruct((n, 2 * h), jnp.float32),
            jax.ShapeDtypeStruct((n, 2 * h), jnp.float32),
            jax.ShapeDtypeStruct((n, 2 * h), jnp.float32),
        ],
    )(sums, x_0, w0cat, W1_on, W1_tg, p1_w, p1b, a, p2_w, p2b, inv_e)

    return (h_on, h_pred, h_tg)
